# baseline (speedup 1.0000x reference)
import functools

import jax
import jax.numpy as jnp
from jax import lax
from jax.experimental import pallas as pl
from jax.experimental.pallas import tpu as pltpu
from jax.experimental.pallas import tpu_sc as plsc

_NUM_FIELDS = 26
_VOCAB = 100000
_D = 16
_BATCH = 16384
_TOTAL = _BATCH * _NUM_FIELDS
_NW = 32
_PER_W = _TOTAL // _NW
_CHUNK = 256
_NCH = _PER_W // _CHUNK
_LANES = 16
_NGRP = _NUM_FIELDS * _VOCAB // 8


def _build():
    mesh = plsc.VectorSubcoreMesh(core_axis_name="c", subcore_axis_name="s")

    @functools.partial(
        pl.kernel,
        mesh=mesh,
        out_type=jax.ShapeDtypeStruct((_TOTAL, _D), jnp.float32),
        scratch_types=[
            pltpu.VMEM((2, _CHUNK), jnp.int32),
            pltpu.VMEM((_CHUNK,), jnp.int32),
            pltpu.VMEM((_CHUNK,), jnp.int32),
            pltpu.VMEM((_CHUNK, 128), jnp.float32),
            pltpu.VMEM((_CHUNK, 128), jnp.float32),
            pltpu.VMEM((_CHUNK, _D), jnp.float32),
            pltpu.SemaphoreType.DMA,
            pltpu.SemaphoreType.DMA,
            pltpu.SemaphoreType.DMA,
            pltpu.SemaphoreType.DMA,
        ],
    )
    def emb(idx_hbm, gidx_hbm, table_hbm, out_hbm,
            idx_v, gidx_a, gidx_b, grp_a, grp_b, rows_v,
            gs0, gs1, isem, osem):
        wid = lax.axis_index("s") * 2 + lax.axis_index("c")
        base = wid * _PER_W

        gidxs = (gidx_a, gidx_b)
        grps = (grp_a, grp_b)
        gsems = (gs0, gs1)

        def istart(c, s):
            pltpu.async_copy(
                idx_hbm.at[pl.ds(base + c * _CHUNK, _CHUNK)], idx_v.at[s], isem)
            pltpu.async_copy(
                gidx_hbm.at[pl.ds(base + c * _CHUNK, _CHUNK)], gidxs[s], isem)

        def iwait(s):
            pltpu.make_async_copy(
                idx_hbm.at[pl.ds(0, _CHUNK)], idx_v.at[s], isem).wait()
            pltpu.make_async_copy(
                idx_hbm.at[pl.ds(0, _CHUNK)], gidxs[s], isem).wait()

        def gstart(s):
            pltpu.async_copy(table_hbm.at[gidxs[s]], grps[s], gsems[s])

        def gwait(s):
            pltpu.make_async_copy(
                table_hbm.at[pl.ds(0, _CHUNK)], grps[s], gsems[s]).wait()

        def ostart(c):
            pltpu.async_copy(
                rows_v, out_hbm.at[pl.ds(base + c * _CHUNK, _CHUNK)], osem)

        def owait():
            pltpu.make_async_copy(
                rows_v, out_hbm.at[pl.ds(base, _CHUNK)], osem).wait()

        def extract(s):
            def body(i, carry):
                subv = (idx_v[s, pl.ds(i * _LANES, _LANES)] & 7) * _LANES
                for l in range(_LANES):
                    r = i * _LANES + l
                    col = pl.multiple_of(subv[l], _LANES)
                    rows_v[r, pl.ds(0, _LANES)] = grps[s][r, pl.ds(col, _LANES)]
                return carry
            lax.fori_loop(0, _CHUNK // _LANES, body, 0)

        istart(0, 0)
        iwait(0)
        gstart(0)
        istart(1, 1)

        npairs = _NCH // 2

        def pair(k, carry):
            c0 = 2 * k
            gwait(0)
            iwait(1)
            gstart(1)

            @pl.when(k > 0)
            def _():
                owait()

            extract(0)
            ostart(c0)

            @pl.when(k < npairs - 1)
            def _():
                istart(c0 + 2, 0)

            gwait(1)

            @pl.when(k < npairs - 1)
            def _():
                iwait(0)
                gstart(0)

            owait()
            extract(1)
            ostart(c0 + 1)

            @pl.when(k < npairs - 1)
            def _():
                istart(c0 + 3, 1)

            return carry

        lax.fori_loop(0, npairs, pair, 0)
        owait()

    return emb


_emb_lookup = _build()


def kernel(x_cat, tables):
    offs = jnp.arange(_NUM_FIELDS, dtype=jnp.int32) * _VOCAB
    flat_idx = (x_cat + offs[None, :]).T.reshape(_TOTAL)
    flat_grp = flat_idx >> 3
    grp_tables = tables.reshape(_NGRP, 128)
    out = _emb_lookup(flat_idx, flat_grp, grp_tables)
    return out.reshape(_NUM_FIELDS, _BATCH, _D).transpose(1, 0, 2)
